# Initial kernel scaffold; baseline (speedup 1.0000x reference)
import jax
import jax.numpy as jnp
from jax.experimental import pallas as pl


def kernel(adj, features, W0, b0, W1, b1, W2, b2):
    raise NotImplementedError("write your pallas kernel here")


# speedup vs baseline: 4.8741x; 4.8741x over previous
import functools

import jax
import jax.numpy as jnp
from jax import lax
from jax.experimental import pallas as pl
from jax.experimental.pallas import tpu as pltpu
from jax.experimental.pallas import tpu_sc as plsc

_NS = 16
_NC = 2


def _row_split(n, sid, f):
    base = (n // _NS) // 8 * 8
    last = n - (_NS - 1) * base

    @pl.when(sid < _NS - 1)
    def _():
        f(pl.multiple_of(sid * base, 8), base)

    @pl.when(sid == _NS - 1)
    def _():
        f((_NS - 1) * base, last)


def _make_degree(n, e):
    ept = e // (_NC * _NS)
    db = 200
    nb = ept // db
    rpt = n // _NS
    mesh = plsc.VectorSubcoreMesh(core_axis_name="c", subcore_axis_name="s")

    @functools.partial(
        pl.kernel, mesh=mesh,
        out_type=[jax.ShapeDtypeStruct((n,), jnp.float32),
                  jax.ShapeDtypeStruct((n,), jnp.float32)],
        scratch_types=[
            pltpu.VMEM((db,), jnp.int32),
            pltpu.VMEM((db,), jnp.float32),
            pltpu.VMEM((16,), jnp.float32),
            pltpu.VMEM_SHARED((n,), jnp.float32),
        ],
    )
    def deg_kernel(dst_h, ones_h, z1, out0, out1, didx, ones_v, stage, dacc):
        cid = lax.axis_index("c")
        sid = lax.axis_index("s")

        def zero(r0, cnt):
            def step(i, carry):
                r = pl.multiple_of(r0 + i * 16, 8)
                pltpu.sync_copy(z1.at[pl.ds(r, 16)], stage)
                pltpu.sync_copy(stage, dacc.at[pl.ds(r, 16)])
                return carry
            lax.fori_loop(0, cnt // 16, step, 0)

        _row_split(n, sid, zero)
        pltpu.sync_copy(ones_h, ones_v)
        plsc.subcore_barrier()

        def blk(b, carry):
            eb = cid * (e // 2) + sid * ept + b * db
            pltpu.sync_copy(dst_h.at[pl.ds(eb, db)], didx)
            pltpu.sync_copy(ones_v, dacc.at[didx], add=True)
            return carry

        lax.fori_loop(0, nb, blk, 0)
        plsc.subcore_barrier()

        def wb(out_ref):
            def f(r0, cnt):
                def step(i, carry):
                    r = pl.multiple_of(r0 + i * 16, 8)
                    pltpu.sync_copy(dacc.at[pl.ds(r, 16)], stage)
                    pltpu.sync_copy(stage, out_ref.at[pl.ds(r, 16)])
                    return carry
                lax.fori_loop(0, cnt // 16, step, 0)
            _row_split(n, sid, f)

        @pl.when(cid == 0)
        def _():
            wb(out0)

        @pl.when(cid == 1)
        def _():
            wb(out1)

    return deg_kernel


def _make_prop(n, e, w, cps):
    ept = e // _NS
    eb_sz = 80
    nb = ept // eb_sz
    rpt = n // _NS
    mesh = plsc.VectorSubcoreMesh(core_axis_name="c", subcore_axis_name="s")
    cparams = ({"compiler_params": pltpu.CompilerParams(
        use_tc_tiling_on_sc=False)} if w < 128 else {})

    @functools.partial(
        pl.kernel, mesh=mesh, **cparams,
        out_type=jax.ShapeDtypeStruct((2 * cps, n, w), jnp.float32),
        scratch_types=[
            pltpu.VMEM((eb_sz,), jnp.int32),
            pltpu.VMEM((eb_sz,), jnp.int32),
            pltpu.VMEM((eb_sz, w), jnp.float32),
            pltpu.VMEM((8, w), jnp.float32),
            pltpu.VMEM_SHARED((n, w), jnp.float32),
            pltpu.SemaphoreType.DMA,
        ],
    )
    def prop_kernel(hs, src_h, dst_h, z, out, sidx, didx, rows, stage, acc,
                    sem):
        cid = lax.axis_index("c")
        sid = lax.axis_index("s")

        def staged(srcf, dstf, r0, cnt):
            def step(i, carry):
                r = pl.multiple_of(r0 + i * 8, 8)
                pltpu.sync_copy(srcf(r, 8), stage.at[pl.ds(0, 8)])
                pltpu.sync_copy(stage.at[pl.ds(0, 8)], dstf(r, 8))
                return carry
            lax.fori_loop(0, cnt // 8, step, 0)

        def chunk_body(ch):
            _row_split(n, sid, lambda r0, cnt: staged(
                lambda r, c: z.at[pl.ds(r, c)],
                lambda r, c: acc.at[pl.ds(r, c)], r0, cnt))
            plsc.subcore_barrier()

            def blk(b, carry):
                eb = sid * ept + b * eb_sz
                pltpu.sync_copy(src_h.at[pl.ds(eb, eb_sz)], sidx)
                pltpu.sync_copy(dst_h.at[pl.ds(eb, eb_sz)], didx)
                pltpu.async_copy(hs.at[ch].at[sidx], rows, sem).wait()
                pltpu.sync_copy(rows, acc.at[didx], add=True)
                return carry

            lax.fori_loop(0, nb, blk, 0)
            plsc.subcore_barrier()
            _row_split(n, sid, lambda r0, cnt: staged(
                lambda r, c: acc.at[pl.ds(r, c)],
                lambda r, c: out.at[ch, pl.ds(r, c)], r0, cnt))

        @pl.when(cid == 0)
        def _():
            for i in range(cps):
                chunk_body(i)

        @pl.when(cid == 1)
        def _():
            for i in range(cps):
                chunk_body(cps + i)

    return prop_kernel


def _dinv_kernel(d0_ref, d1_ref, out_ref):
    s = d0_ref[...] + d1_ref[...] + 1.0
    out_ref[...] = lax.rsqrt(jnp.maximum(s, 1.0))


def _dinv(dp0, dp1, n, mb=2000):
    return pl.pallas_call(
        _dinv_kernel,
        grid=(n // mb,),
        in_specs=[pl.BlockSpec((mb, 1), lambda m: (m, 0)),
                  pl.BlockSpec((mb, 1), lambda m: (m, 0))],
        out_specs=pl.BlockSpec((mb, 1), lambda m: (m, 0)),
        out_shape=jax.ShapeDtypeStruct((n, 1), jnp.float32),
    )(dp0, dp1)


def _mm0_kernel(x_ref, w_ref, dinv_ref, out_ref):
    h = jnp.dot(x_ref[...], w_ref[...], preferred_element_type=jnp.float32)
    out_ref[0] = h * dinv_ref[...]


def _mm0(x, w0, dinv, n, din, mb=2000):
    return pl.pallas_call(
        _mm0_kernel,
        grid=(n // mb, 4),
        in_specs=[
            pl.BlockSpec((mb, din), lambda m, c: (m, 0)),
            pl.BlockSpec((din, 128), lambda m, c: (0, c)),
            pl.BlockSpec((mb, 1), lambda m, c: (m, 0)),
        ],
        out_specs=pl.BlockSpec((1, mb, 128), lambda m, c: (c, m, 0)),
        out_shape=jax.ShapeDtypeStruct((4, n, 128), jnp.float32),
        compiler_params=pltpu.CompilerParams(
            dimension_semantics=("parallel", "parallel")),
    )(x, w0, dinv)


def _mm_mid_kernel(kc_last, p_ref, hs_ref, b_ref, dinv_ref, w_ref, out_ref):
    kc = pl.program_id(2)
    x = dinv_ref[...] * (p_ref[0] + hs_ref[0]) + b_ref[0]
    x = jnp.maximum(x, 0.0)
    part = jnp.dot(x, w_ref[...], preferred_element_type=jnp.float32)

    @pl.when(kc == 0)
    def _():
        out_ref[0] = part

    @pl.when(kc != 0)
    def _():
        out_ref[0] += part

    @pl.when(kc == kc_last)
    def _():
        out_ref[0] *= dinv_ref[...]


def _mm_mid(p, hs, b, dinv, w, n, mb=2000):
    cout = w.shape[1] // 128
    kcn = w.shape[0] // 128
    return pl.pallas_call(
        functools.partial(_mm_mid_kernel, kcn - 1),
        grid=(n // mb, cout, kcn),
        in_specs=[
            pl.BlockSpec((1, mb, 128), lambda m, c, kc: (kc, m, 0)),
            pl.BlockSpec((1, mb, 128), lambda m, c, kc: (kc, m, 0)),
            pl.BlockSpec((1, 1, 128), lambda m, c, kc: (kc, 0, 0)),
            pl.BlockSpec((mb, 1), lambda m, c, kc: (m, 0)),
            pl.BlockSpec((128, 128), lambda m, c, kc: (kc, c)),
        ],
        out_specs=pl.BlockSpec((1, mb, 128), lambda m, c, kc: (c, m, 0)),
        out_shape=jax.ShapeDtypeStruct((cout, n, 128), jnp.float32),
        compiler_params=pltpu.CompilerParams(
            dimension_semantics=("parallel", "parallel", "arbitrary")),
    )(p, hs, b, dinv, w)


def _mm_narrow_kernel(kc_last, p_ref, hs_ref, b_ref, dinv_ref, w_ref, out_ref):
    kc = pl.program_id(1)
    x = dinv_ref[...] * (p_ref[0] + hs_ref[0]) + b_ref[0]
    x = jnp.maximum(x, 0.0)
    part = jnp.dot(x, w_ref[...], preferred_element_type=jnp.float32)

    @pl.when(kc == 0)
    def _():
        out_ref[0] = part[:, :32]
        out_ref[1] = part[:, 32:]

    @pl.when(kc != 0)
    def _():
        out_ref[0] += part[:, :32]
        out_ref[1] += part[:, 32:]

    @pl.when(kc == kc_last)
    def _():
        out_ref[0] *= dinv_ref[...]
        out_ref[1] *= dinv_ref[...]


def _mm_narrow(p, hs, b, dinv, w, n, mb=2000):
    kcn = w.shape[0] // 128
    return pl.pallas_call(
        functools.partial(_mm_narrow_kernel, kcn - 1),
        grid=(n // mb, kcn),
        in_specs=[
            pl.BlockSpec((1, mb, 128), lambda m, kc: (kc, m, 0)),
            pl.BlockSpec((1, mb, 128), lambda m, kc: (kc, m, 0)),
            pl.BlockSpec((1, 1, 128), lambda m, kc: (kc, 0, 0)),
            pl.BlockSpec((mb, 1), lambda m, kc: (m, 0)),
            pl.BlockSpec((128, 64), lambda m, kc: (kc, 0)),
        ],
        out_specs=pl.BlockSpec((2, mb, 32), lambda m, kc: (0, m, 0)),
        out_shape=jax.ShapeDtypeStruct((2, n, 32), jnp.float32),
        compiler_params=pltpu.CompilerParams(
            dimension_semantics=("parallel", "arbitrary")),
    )(p, hs, b, dinv, w)


def _final_kernel(p_ref, hs_ref, b_ref, dinv_ref, out_ref):
    d = dinv_ref[...]
    t0 = d * (p_ref[0] + hs_ref[0]) + b_ref[0]
    t1 = d * (p_ref[1] + hs_ref[1]) + b_ref[1]
    out_ref[...] = jnp.concatenate([t0, t1], axis=1)


def _final(p, hs, b, dinv, n, mb=2000):
    return pl.pallas_call(
        _final_kernel,
        grid=(n // mb,),
        in_specs=[
            pl.BlockSpec((2, mb, 32), lambda m: (0, m, 0)),
            pl.BlockSpec((2, mb, 32), lambda m: (0, m, 0)),
            pl.BlockSpec((2, 1, 32), lambda m: (0, 0, 0)),
            pl.BlockSpec((mb, 1), lambda m: (m, 0)),
        ],
        out_specs=pl.BlockSpec((mb, 64), lambda m: (m, 0)),
        out_shape=jax.ShapeDtypeStruct((n, 64), jnp.float32),
    )(p, hs, b, dinv)


def kernel(adj, features, W0, b0, W1, b1, W2, b2):
    n, din = features.shape
    e = adj.shape[1]
    hid = W1.shape[0]
    ncls = W2.shape[1]

    ones_h = jnp.ones((200,), jnp.float32)
    z1 = jnp.zeros((n,), jnp.float32)
    z128 = jnp.zeros((n, 128), jnp.float32)
    z32 = jnp.zeros((n, 32), jnp.float32)

    src = adj[0]
    dst = adj[1]
    dp0, dp1 = _make_degree(n, e)(dst, ones_h, z1)
    dinv = _dinv(dp0.reshape(n, 1), dp1.reshape(n, 1), n)

    prop_wide = _make_prop(n, e, 128, 2)
    prop_narrow = _make_prop(n, e, 32, 1)

    hs0 = _mm0(features, W0, dinv, n, din)
    p0 = prop_wide(hs0, src, dst, z128)
    hs1 = _mm_mid(p0, hs0, b0.reshape(hid // 128, 1, 128), dinv, W1, n)
    p1 = prop_wide(hs1, src, dst, z128)
    hs2 = _mm_narrow(p1, hs1, b1.reshape(hid // 128, 1, 128), dinv, W2, n)
    p2 = prop_narrow(hs2, src, dst, z32)
    out = _final(p2, hs2, b2.reshape(2, 1, 32), dinv, n)
    return out


# speedup vs baseline: 7.9371x; 1.6284x over previous
import functools

import jax
import jax.numpy as jnp
from jax import lax
from jax.experimental import pallas as pl
from jax.experimental.pallas import tpu as pltpu
from jax.experimental.pallas import tpu_sc as plsc

_NS = 16
_NC = 2


def _row_split(n, sid, f):
    base = (n // _NS) // 8 * 8
    last = n - (_NS - 1) * base

    @pl.when(sid < _NS - 1)
    def _():
        f(pl.multiple_of(sid * base, 8), base)

    @pl.when(sid == _NS - 1)
    def _():
        f((_NS - 1) * base, last)


def _make_degree(n, e):
    ept = e // (_NC * _NS)
    db = 200
    nb = ept // db
    rpt = n // _NS
    mesh = plsc.VectorSubcoreMesh(core_axis_name="c", subcore_axis_name="s")

    @functools.partial(
        pl.kernel, mesh=mesh,
        out_type=[jax.ShapeDtypeStruct((n,), jnp.float32),
                  jax.ShapeDtypeStruct((n,), jnp.float32)],
        scratch_types=[
            pltpu.VMEM((db,), jnp.int32),
            pltpu.VMEM((db,), jnp.float32),
            pltpu.VMEM((16,), jnp.float32),
            pltpu.VMEM_SHARED((n,), jnp.float32),
        ],
    )
    def deg_kernel(dst_h, ones_h, z1, out0, out1, didx, ones_v, stage, dacc):
        cid = lax.axis_index("c")
        sid = lax.axis_index("s")

        def zero(r0, cnt):
            def step(i, carry):
                r = pl.multiple_of(r0 + i * 16, 8)
                pltpu.sync_copy(z1.at[pl.ds(r, 16)], stage)
                pltpu.sync_copy(stage, dacc.at[pl.ds(r, 16)])
                return carry
            lax.fori_loop(0, cnt // 16, step, 0)

        _row_split(n, sid, zero)
        pltpu.sync_copy(ones_h, ones_v)
        plsc.subcore_barrier()

        def blk(b, carry):
            eb = cid * (e // 2) + sid * ept + b * db
            pltpu.sync_copy(dst_h.at[pl.ds(eb, db)], didx)
            pltpu.sync_copy(ones_v, dacc.at[didx], add=True)
            return carry

        lax.fori_loop(0, nb, blk, 0)
        plsc.subcore_barrier()

        def wb(out_ref):
            def f(r0, cnt):
                def step(i, carry):
                    r = pl.multiple_of(r0 + i * 16, 8)
                    pltpu.sync_copy(dacc.at[pl.ds(r, 16)], stage)
                    pltpu.sync_copy(stage, out_ref.at[pl.ds(r, 16)])
                    return carry
                lax.fori_loop(0, cnt // 16, step, 0)
            _row_split(n, sid, f)

        @pl.when(cid == 0)
        def _():
            wb(out0)

        @pl.when(cid == 1)
        def _():
            wb(out1)

    return deg_kernel


def _make_prop(n, e, w, cps):
    ept = e // _NS
    eb_sz = 80 if w >= 128 else 200
    nb = ept // eb_sz
    rpt = n // _NS
    mesh = plsc.VectorSubcoreMesh(core_axis_name="c", subcore_axis_name="s")
    cparams = ({"compiler_params": pltpu.CompilerParams(
        use_tc_tiling_on_sc=False)} if w < 128 else {})

    @functools.partial(
        pl.kernel, mesh=mesh, **cparams,
        out_type=jax.ShapeDtypeStruct((2 * cps, n, w), jnp.float32),
        scratch_types=[
            pltpu.VMEM((eb_sz,), jnp.int32),
            pltpu.VMEM((eb_sz,), jnp.int32),
            pltpu.VMEM((eb_sz,), jnp.int32),
            pltpu.VMEM((eb_sz,), jnp.int32),
            pltpu.VMEM((eb_sz, w), jnp.float32),
            pltpu.VMEM((eb_sz, w), jnp.float32),
            pltpu.VMEM((8, w), jnp.float32),
            pltpu.VMEM_SHARED((n, w), jnp.float32),
            pltpu.SemaphoreType.DMA,
            pltpu.SemaphoreType.DMA,
            pltpu.SemaphoreType.DMA,
            pltpu.SemaphoreType.DMA,
            pltpu.SemaphoreType.DMA,
            pltpu.SemaphoreType.DMA,
        ],
    )
    def prop_kernel(hs, src_h, dst_h, z, out, sidx0, sidx1, didx0, didx1,
                    rows0, rows1, stage, acc,
                    isem0, isem1, gsem0, gsem1, ssem0, ssem1):
        cid = lax.axis_index("c")
        sid = lax.axis_index("s")
        sidx = (sidx0, sidx1)
        didx = (didx0, didx1)
        rows = (rows0, rows1)
        isem = (isem0, isem1)
        gsem = (gsem0, gsem1)
        ssem = (ssem0, ssem1)

        def idx_load(b, p):
            eb = sid * ept + b * eb_sz
            pltpu.async_copy(src_h.at[pl.ds(eb, eb_sz)], sidx[p], isem[p])
            pltpu.async_copy(dst_h.at[pl.ds(eb, eb_sz)], didx[p], isem[p])

        def idx_wait(p):
            pltpu.make_async_copy(src_h.at[pl.ds(0, eb_sz)], sidx[p],
                                  isem[p]).wait()
            pltpu.make_async_copy(dst_h.at[pl.ds(0, eb_sz)], didx[p],
                                  isem[p]).wait()

        def staged(srcf, dstf, r0, cnt):
            def step(i, carry):
                r = pl.multiple_of(r0 + i * 8, 8)
                pltpu.sync_copy(srcf(r, 8), stage.at[pl.ds(0, 8)])
                pltpu.sync_copy(stage.at[pl.ds(0, 8)], dstf(r, 8))
                return carry
            lax.fori_loop(0, cnt // 8, step, 0)

        def chunk_body(ch):
            _row_split(n, sid, lambda r0, cnt: staged(
                lambda r, c: z.at[pl.ds(r, c)],
                lambda r, c: acc.at[pl.ds(r, c)], r0, cnt))
            plsc.subcore_barrier()

            idx_load(0, 0)

            def blk2(h, carry):
                for q in range(2):
                    b = h * 2 + q

                    @pl.when(b < nb)
                    def _():
                        idx_wait(q)

                        @pl.when(b + 1 < nb)
                        def _():
                            idx_load(b + 1, 1 - q)

                        @pl.when(b >= 2)
                        def _():
                            pltpu.make_async_copy(
                                rows[q], acc.at[didx[q]], ssem[q]).wait()
                        pltpu.async_copy(hs.at[ch].at[sidx[q]], rows[q],
                                         gsem[q])
                        pltpu.make_async_copy(hs.at[ch].at[sidx[q]], rows[q],
                                              gsem[q]).wait()
                        pltpu.async_copy(rows[q], acc.at[didx[q]], ssem[q],
                                         add=True)
                return carry

            lax.fori_loop(0, (nb + 1) // 2, blk2, 0)
            for q in range(2):
                @pl.when(nb > q)
                def _():
                    pltpu.make_async_copy(rows[q], acc.at[didx[q]],
                                          ssem[q]).wait()
            plsc.subcore_barrier()
            _row_split(n, sid, lambda r0, cnt: staged(
                lambda r, c: acc.at[pl.ds(r, c)],
                lambda r, c: out.at[ch, pl.ds(r, c)], r0, cnt))

        @pl.when(cid == 0)
        def _():
            for i in range(cps):
                chunk_body(i)

        @pl.when(cid == 1)
        def _():
            for i in range(cps):
                chunk_body(cps + i)

    return prop_kernel


def _dinv_kernel(d0_ref, d1_ref, out_ref):
    s = d0_ref[...] + d1_ref[...] + 1.0
    out_ref[...] = lax.rsqrt(jnp.maximum(s, 1.0))


def _dinv(dp0, dp1, n, mb=2000):
    return pl.pallas_call(
        _dinv_kernel,
        grid=(n // mb,),
        in_specs=[pl.BlockSpec((mb, 1), lambda m: (m, 0)),
                  pl.BlockSpec((mb, 1), lambda m: (m, 0))],
        out_specs=pl.BlockSpec((mb, 1), lambda m: (m, 0)),
        out_shape=jax.ShapeDtypeStruct((n, 1), jnp.float32),
    )(dp0, dp1)


def _mm0_kernel(x_ref, w_ref, dinv_ref, out_ref):
    h = jnp.dot(x_ref[...], w_ref[...], preferred_element_type=jnp.float32)
    out_ref[0] = h * dinv_ref[...]


def _mm0(x, w0, dinv, n, din, mb=2000):
    return pl.pallas_call(
        _mm0_kernel,
        grid=(n // mb, 4),
        in_specs=[
            pl.BlockSpec((mb, din), lambda m, c: (m, 0)),
            pl.BlockSpec((din, 128), lambda m, c: (0, c)),
            pl.BlockSpec((mb, 1), lambda m, c: (m, 0)),
        ],
        out_specs=pl.BlockSpec((1, mb, 128), lambda m, c: (c, m, 0)),
        out_shape=jax.ShapeDtypeStruct((4, n, 128), jnp.float32),
        compiler_params=pltpu.CompilerParams(
            dimension_semantics=("parallel", "parallel")),
    )(x, w0, dinv)


def _mm_mid_kernel(kc_last, p_ref, hs_ref, b_ref, dinv_ref, w_ref, out_ref):
    kc = pl.program_id(2)
    x = dinv_ref[...] * (p_ref[0] + hs_ref[0]) + b_ref[0]
    x = jnp.maximum(x, 0.0)
    part = jnp.dot(x, w_ref[...], preferred_element_type=jnp.float32)

    @pl.when(kc == 0)
    def _():
        out_ref[0] = part

    @pl.when(kc != 0)
    def _():
        out_ref[0] += part

    @pl.when(kc == kc_last)
    def _():
        out_ref[0] *= dinv_ref[...]


def _mm_mid(p, hs, b, dinv, w, n, mb=2000):
    cout = w.shape[1] // 128
    kcn = w.shape[0] // 128
    return pl.pallas_call(
        functools.partial(_mm_mid_kernel, kcn - 1),
        grid=(n // mb, cout, kcn),
        in_specs=[
            pl.BlockSpec((1, mb, 128), lambda m, c, kc: (kc, m, 0)),
            pl.BlockSpec((1, mb, 128), lambda m, c, kc: (kc, m, 0)),
            pl.BlockSpec((1, 1, 128), lambda m, c, kc: (kc, 0, 0)),
            pl.BlockSpec((mb, 1), lambda m, c, kc: (m, 0)),
            pl.BlockSpec((128, 128), lambda m, c, kc: (kc, c)),
        ],
        out_specs=pl.BlockSpec((1, mb, 128), lambda m, c, kc: (c, m, 0)),
        out_shape=jax.ShapeDtypeStruct((cout, n, 128), jnp.float32),
        compiler_params=pltpu.CompilerParams(
            dimension_semantics=("parallel", "parallel", "arbitrary")),
    )(p, hs, b, dinv, w)


def _mm_narrow_kernel(kc_last, p_ref, hs_ref, b_ref, dinv_ref, w_ref, out_ref):
    kc = pl.program_id(1)
    x = dinv_ref[...] * (p_ref[0] + hs_ref[0]) + b_ref[0]
    x = jnp.maximum(x, 0.0)
    part = jnp.dot(x, w_ref[...], preferred_element_type=jnp.float32)

    @pl.when(kc == 0)
    def _():
        out_ref[0] = part[:, :32]
        out_ref[1] = part[:, 32:]

    @pl.when(kc != 0)
    def _():
        out_ref[0] += part[:, :32]
        out_ref[1] += part[:, 32:]

    @pl.when(kc == kc_last)
    def _():
        out_ref[0] *= dinv_ref[...]
        out_ref[1] *= dinv_ref[...]


def _mm_narrow(p, hs, b, dinv, w, n, mb=2000):
    kcn = w.shape[0] // 128
    return pl.pallas_call(
        functools.partial(_mm_narrow_kernel, kcn - 1),
        grid=(n // mb, kcn),
        in_specs=[
            pl.BlockSpec((1, mb, 128), lambda m, kc: (kc, m, 0)),
            pl.BlockSpec((1, mb, 128), lambda m, kc: (kc, m, 0)),
            pl.BlockSpec((1, 1, 128), lambda m, kc: (kc, 0, 0)),
            pl.BlockSpec((mb, 1), lambda m, kc: (m, 0)),
            pl.BlockSpec((128, 64), lambda m, kc: (kc, 0)),
        ],
        out_specs=pl.BlockSpec((2, mb, 32), lambda m, kc: (0, m, 0)),
        out_shape=jax.ShapeDtypeStruct((2, n, 32), jnp.float32),
        compiler_params=pltpu.CompilerParams(
            dimension_semantics=("parallel", "arbitrary")),
    )(p, hs, b, dinv, w)


def _final_kernel(p_ref, hs_ref, b_ref, dinv_ref, out_ref):
    d = dinv_ref[...]
    t0 = d * (p_ref[0] + hs_ref[0]) + b_ref[0]
    t1 = d * (p_ref[1] + hs_ref[1]) + b_ref[1]
    out_ref[...] = jnp.concatenate([t0, t1], axis=1)


def _final(p, hs, b, dinv, n, mb=2000):
    return pl.pallas_call(
        _final_kernel,
        grid=(n // mb,),
        in_specs=[
            pl.BlockSpec((2, mb, 32), lambda m: (0, m, 0)),
            pl.BlockSpec((2, mb, 32), lambda m: (0, m, 0)),
            pl.BlockSpec((2, 1, 32), lambda m: (0, 0, 0)),
            pl.BlockSpec((mb, 1), lambda m: (m, 0)),
        ],
        out_specs=pl.BlockSpec((mb, 64), lambda m: (m, 0)),
        out_shape=jax.ShapeDtypeStruct((n, 64), jnp.float32),
    )(p, hs, b, dinv)


def kernel(adj, features, W0, b0, W1, b1, W2, b2):
    n, din = features.shape
    e = adj.shape[1]
    hid = W1.shape[0]
    ncls = W2.shape[1]

    ones_h = jnp.ones((200,), jnp.float32)
    z1 = jnp.zeros((n,), jnp.float32)
    z128 = jnp.zeros((n, 128), jnp.float32)
    z32 = jnp.zeros((n, 32), jnp.float32)

    src = adj[0]
    dst = adj[1]
    dp0, dp1 = _make_degree(n, e)(dst, ones_h, z1)
    dinv = _dinv(dp0.reshape(n, 1), dp1.reshape(n, 1), n)

    prop_wide = _make_prop(n, e, 128, 2)
    prop_narrow = _make_prop(n, e, 32, 1)

    hs0 = _mm0(features, W0, dinv, n, din)
    p0 = prop_wide(hs0, src, dst, z128)
    hs1 = _mm_mid(p0, hs0, b0.reshape(hid // 128, 1, 128), dinv, W1, n)
    p1 = prop_wide(hs1, src, dst, z128)
    hs2 = _mm_narrow(p1, hs1, b1.reshape(hid // 128, 1, 128), dinv, W2, n)
    p2 = prop_narrow(hs2, src, dst, z32)
    out = _final(p2, hs2, b2.reshape(2, 1, 32), dinv, n)
    return out


# speedup vs baseline: 33.5814x; 4.2310x over previous
import functools

import jax
import jax.numpy as jnp
from jax import lax
from jax.experimental import pallas as pl
from jax.experimental.pallas import tpu as pltpu
from jax.experimental.pallas import tpu_sc as plsc

_NS = 16
_NC = 2


def _row_split(n, sid, f):
    base = (n // _NS) // 8 * 8
    last = n - (_NS - 1) * base

    @pl.when(sid < _NS - 1)
    def _():
        f(pl.multiple_of(sid * base, 8), base)

    @pl.when(sid == _NS - 1)
    def _():
        f((_NS - 1) * base, last)


def _make_degree(n, e):
    ept = e // (_NC * _NS)
    db = 200
    nb = ept // db
    rpt = n // _NS
    mesh = plsc.VectorSubcoreMesh(core_axis_name="c", subcore_axis_name="s")

    @functools.partial(
        pl.kernel, mesh=mesh,
        out_type=[jax.ShapeDtypeStruct((n,), jnp.float32),
                  jax.ShapeDtypeStruct((n,), jnp.float32)],
        scratch_types=[
            pltpu.VMEM((db,), jnp.int32),
            pltpu.VMEM((db,), jnp.float32),
            pltpu.VMEM((16,), jnp.float32),
            pltpu.VMEM_SHARED((n,), jnp.float32),
        ],
    )
    def deg_kernel(dst_h, ones_h, z1, out0, out1, didx, ones_v, stage, dacc):
        cid = lax.axis_index("c")
        sid = lax.axis_index("s")

        def zero(r0, cnt):
            def step(i, carry):
                r = pl.multiple_of(r0 + i * 16, 8)
                pltpu.sync_copy(z1.at[pl.ds(r, 16)], stage)
                pltpu.sync_copy(stage, dacc.at[pl.ds(r, 16)])
                return carry
            lax.fori_loop(0, cnt // 16, step, 0)

        _row_split(n, sid, zero)
        pltpu.sync_copy(ones_h, ones_v)
        plsc.subcore_barrier()

        def blk(b, carry):
            eb = cid * (e // 2) + sid * ept + b * db
            pltpu.sync_copy(dst_h.at[pl.ds(eb, db)], didx)
            pltpu.sync_copy(ones_v, dacc.at[didx], add=True)
            return carry

        lax.fori_loop(0, nb, blk, 0)
        plsc.subcore_barrier()

        def wb(out_ref):
            def f(r0, cnt):
                def step(i, carry):
                    r = pl.multiple_of(r0 + i * 16, 8)
                    pltpu.sync_copy(dacc.at[pl.ds(r, 16)], stage)
                    pltpu.sync_copy(stage, out_ref.at[pl.ds(r, 16)])
                    return carry
                lax.fori_loop(0, cnt // 16, step, 0)
            _row_split(n, sid, f)

        @pl.when(cid == 0)
        def _():
            wb(out0)

        @pl.when(cid == 1)
        def _():
            wb(out1)

    return deg_kernel


def _make_prop(n, e, w, cps):
    ept = e // _NS
    eb_sz = 80 if w >= 128 else 200
    nb = ept // eb_sz
    rpt = n // _NS
    mesh = plsc.VectorSubcoreMesh(core_axis_name="c", subcore_axis_name="s")
    cparams = ({"compiler_params": pltpu.CompilerParams(
        use_tc_tiling_on_sc=False)} if w < 128 else {})

    @functools.partial(
        pl.kernel, mesh=mesh, **cparams,
        out_type=jax.ShapeDtypeStruct((2 * cps, n, w), jnp.float32),
        scratch_types=(
            [pltpu.VMEM((eb_sz,), jnp.int32) for _ in range(8)] +
            [pltpu.VMEM((eb_sz, w), jnp.float32) for _ in range(2)] +
            [pltpu.VMEM((8, w), jnp.float32),
             pltpu.VMEM_SHARED((n, w), jnp.float32)] +
            [pltpu.SemaphoreType.DMA for _ in range(8)]),
    )
    def prop_kernel(hs, src_h, dst_h, z, out,
                    si0, si1, si2, si3, di0, di1, di2, di3,
                    rows0, rows1, stage, acc,
                    is0, is1, is2, is3, gsem0, gsem1, ssem0, ssem1):
        cid = lax.axis_index("c")
        sid = lax.axis_index("s")
        sidx = (si0, si1, si2, si3)
        didx = (di0, di1, di2, di3)
        rows = (rows0, rows1)
        isem = (is0, is1, is2, is3)
        gsem = (gsem0, gsem1)
        ssem = (ssem0, ssem1)

        def idx_load(b, slot):
            eb = sid * ept + b * eb_sz
            pltpu.async_copy(src_h.at[pl.ds(eb, eb_sz)], sidx[slot],
                             isem[slot])
            pltpu.async_copy(dst_h.at[pl.ds(eb, eb_sz)], didx[slot],
                             isem[slot])

        def idx_wait(slot):
            pltpu.make_async_copy(src_h.at[pl.ds(0, eb_sz)], sidx[slot],
                                  isem[slot]).wait()
            pltpu.make_async_copy(dst_h.at[pl.ds(0, eb_sz)], didx[slot],
                                  isem[slot]).wait()

        def scat_wait(q):
            pltpu.make_async_copy(rows[q], acc.at[didx[0]], ssem[q]).wait()

        def staged(srcf, dstf, r0, cnt):
            def step(i, carry):
                r = pl.multiple_of(r0 + i * 8, 8)
                pltpu.sync_copy(srcf(r, 8), stage.at[pl.ds(0, 8)])
                pltpu.sync_copy(stage.at[pl.ds(0, 8)], dstf(r, 8))
                return carry
            lax.fori_loop(0, cnt // 8, step, 0)

        def chunk_body(ch):
            _row_split(n, sid, lambda r0, cnt: staged(
                lambda r, c: z.at[pl.ds(r, c)],
                lambda r, c: acc.at[pl.ds(r, c)], r0, cnt))
            plsc.subcore_barrier()

            idx_load(0, 0)
            idx_load(1, 1)
            idx_load(2, 2)
            idx_wait(0)
            pltpu.async_copy(hs.at[ch].at[sidx[0]], rows[0], gsem[0])

            def quad(h, carry):
                for qq in range(4):
                    b = h * 4 + 1 + qq
                    q = (1 + qq) % 2
                    slot = (1 + qq) % 4
                    pslot = qq % 4
                    nslot = (3 + qq) % 4

                    @pl.when(b < nb)
                    def _():
                        idx_wait(slot)

                        @pl.when(b >= 2)
                        def _():
                            scat_wait(q)
                        pltpu.async_copy(hs.at[ch].at[sidx[slot]], rows[q],
                                         gsem[q])

                        @pl.when(b + 2 < nb)
                        def _():
                            idx_load(b + 2, nslot)
                        pltpu.make_async_copy(hs.at[ch].at[sidx[slot]],
                                              rows[1 - q], gsem[1 - q]).wait()
                        pltpu.async_copy(rows[1 - q], acc.at[didx[pslot]],
                                         ssem[1 - q], add=True)
                return carry

            lax.fori_loop(0, (nb + 2) // 4, quad, 0)
            lq = (nb - 1) % 2
            pltpu.make_async_copy(hs.at[ch].at[sidx[0]], rows[lq],
                                  gsem[lq]).wait()
            pltpu.async_copy(rows[lq], acc.at[didx[(nb - 1) % 4]], ssem[lq],
                             add=True)
            scat_wait(0)
            scat_wait(1)
            plsc.subcore_barrier()
            _row_split(n, sid, lambda r0, cnt: staged(
                lambda r, c: acc.at[pl.ds(r, c)],
                lambda r, c: out.at[ch, pl.ds(r, c)], r0, cnt))

    return prop_kernel


def _dinv_kernel(d0_ref, d1_ref, out_ref):
    s = d0_ref[...] + d1_ref[...] + 1.0
    out_ref[...] = lax.rsqrt(jnp.maximum(s, 1.0))


def _dinv(dp0, dp1, n, mb=2000):
    return pl.pallas_call(
        _dinv_kernel,
        grid=(n // mb,),
        in_specs=[pl.BlockSpec((mb, 1), lambda m: (m, 0)),
                  pl.BlockSpec((mb, 1), lambda m: (m, 0))],
        out_specs=pl.BlockSpec((mb, 1), lambda m: (m, 0)),
        out_shape=jax.ShapeDtypeStruct((n, 1), jnp.float32),
    )(dp0, dp1)


def _mm0_kernel(x_ref, w_ref, dinv_ref, out_ref):
    h = jnp.dot(x_ref[...], w_ref[...], preferred_element_type=jnp.float32)
    out_ref[0] = h * dinv_ref[...]


def _mm0(x, w0, dinv, n, din, mb=2000):
    return pl.pallas_call(
        _mm0_kernel,
        grid=(n // mb, 4),
        in_specs=[
            pl.BlockSpec((mb, din), lambda m, c: (m, 0)),
            pl.BlockSpec((din, 128), lambda m, c: (0, c)),
            pl.BlockSpec((mb, 1), lambda m, c: (m, 0)),
        ],
        out_specs=pl.BlockSpec((1, mb, 128), lambda m, c: (c, m, 0)),
        out_shape=jax.ShapeDtypeStruct((4, n, 128), jnp.float32),
        compiler_params=pltpu.CompilerParams(
            dimension_semantics=("parallel", "parallel")),
    )(x, w0, dinv)


def _mm_mid_kernel(kc_last, p_ref, hs_ref, b_ref, dinv_ref, w_ref, out_ref):
    kc = pl.program_id(2)
    x = dinv_ref[...] * (p_ref[0] + hs_ref[0]) + b_ref[0]
    x = jnp.maximum(x, 0.0)
    part = jnp.dot(x, w_ref[...], preferred_element_type=jnp.float32)

    @pl.when(kc == 0)
    def _():
        out_ref[0] = part

    @pl.when(kc != 0)
    def _():
        out_ref[0] += part

    @pl.when(kc == kc_last)
    def _():
        out_ref[0] *= dinv_ref[...]


def _mm_mid(p, hs, b, dinv, w, n, mb=2000):
    cout = w.shape[1] // 128
    kcn = w.shape[0] // 128
    return pl.pallas_call(
        functools.partial(_mm_mid_kernel, kcn - 1),
        grid=(n // mb, cout, kcn),
        in_specs=[
            pl.BlockSpec((1, mb, 128), lambda m, c, kc: (kc, m, 0)),
            pl.BlockSpec((1, mb, 128), lambda m, c, kc: (kc, m, 0)),
            pl.BlockSpec((1, 1, 128), lambda m, c, kc: (kc, 0, 0)),
            pl.BlockSpec((mb, 1), lambda m, c, kc: (m, 0)),
            pl.BlockSpec((128, 128), lambda m, c, kc: (kc, c)),
        ],
        out_specs=pl.BlockSpec((1, mb, 128), lambda m, c, kc: (c, m, 0)),
        out_shape=jax.ShapeDtypeStruct((cout, n, 128), jnp.float32),
        compiler_params=pltpu.CompilerParams(
            dimension_semantics=("parallel", "parallel", "arbitrary")),
    )(p, hs, b, dinv, w)


def _mm_narrow_kernel(kc_last, p_ref, hs_ref, b_ref, dinv_ref, w_ref, out_ref):
    kc = pl.program_id(1)
    x = dinv_ref[...] * (p_ref[0] + hs_ref[0]) + b_ref[0]
    x = jnp.maximum(x, 0.0)
    part = jnp.dot(x, w_ref[...], preferred_element_type=jnp.float32)

    @pl.when(kc == 0)
    def _():
        out_ref[0] = part[:, :32]
        out_ref[1] = part[:, 32:]

    @pl.when(kc != 0)
    def _():
        out_ref[0] += part[:, :32]
        out_ref[1] += part[:, 32:]

    @pl.when(kc == kc_last)
    def _():
        out_ref[0] *= dinv_ref[...]
        out_ref[1] *= dinv_ref[...]


def _mm_narrow(p, hs, b, dinv, w, n, mb=2000):
    kcn = w.shape[0] // 128
    return pl.pallas_call(
        functools.partial(_mm_narrow_kernel, kcn - 1),
        grid=(n // mb, kcn),
        in_specs=[
            pl.BlockSpec((1, mb, 128), lambda m, kc: (kc, m, 0)),
            pl.BlockSpec((1, mb, 128), lambda m, kc: (kc, m, 0)),
            pl.BlockSpec((1, 1, 128), lambda m, kc: (kc, 0, 0)),
            pl.BlockSpec((mb, 1), lambda m, kc: (m, 0)),
            pl.BlockSpec((128, 64), lambda m, kc: (kc, 0)),
        ],
        out_specs=pl.BlockSpec((2, mb, 32), lambda m, kc: (0, m, 0)),
        out_shape=jax.ShapeDtypeStruct((2, n, 32), jnp.float32),
        compiler_params=pltpu.CompilerParams(
            dimension_semantics=("parallel", "arbitrary")),
    )(p, hs, b, dinv, w)


def _final_kernel(p_ref, hs_ref, b_ref, dinv_ref, out_ref):
    d = dinv_ref[...]
    t0 = d * (p_ref[0] + hs_ref[0]) + b_ref[0]
    t1 = d * (p_ref[1] + hs_ref[1]) + b_ref[1]
    out_ref[...] = jnp.concatenate([t0, t1], axis=1)


def _final(p, hs, b, dinv, n, mb=2000):
    return pl.pallas_call(
        _final_kernel,
        grid=(n // mb,),
        in_specs=[
            pl.BlockSpec((2, mb, 32), lambda m: (0, m, 0)),
            pl.BlockSpec((2, mb, 32), lambda m: (0, m, 0)),
            pl.BlockSpec((2, 1, 32), lambda m: (0, 0, 0)),
            pl.BlockSpec((mb, 1), lambda m: (m, 0)),
        ],
        out_specs=pl.BlockSpec((mb, 64), lambda m: (m, 0)),
        out_shape=jax.ShapeDtypeStruct((n, 64), jnp.float32),
    )(p, hs, b, dinv)


def kernel(adj, features, W0, b0, W1, b1, W2, b2):
    n, din = features.shape
    e = adj.shape[1]
    hid = W1.shape[0]
    ncls = W2.shape[1]

    ones_h = jnp.ones((200,), jnp.float32)
    z1 = jnp.zeros((n,), jnp.float32)
    z128 = jnp.zeros((n, 128), jnp.float32)
    z32 = jnp.zeros((n, 32), jnp.float32)

    src = adj[0]
    dst = adj[1]
    dp0, dp1 = _make_degree(n, e)(dst, ones_h, z1)
    dinv = _dinv(dp0.reshape(n, 1), dp1.reshape(n, 1), n)

    prop_wide = _make_prop(n, e, 128, 2)
    prop_narrow = _make_prop(n, e, 32, 1)

    hs0 = _mm0(features, W0, dinv, n, din)
    p0 = prop_wide(hs0, src, dst, z128)
    hs1 = _mm_mid(p0, hs0, b0.reshape(hid // 128, 1, 128), dinv, W1, n)
    p1 = prop_wide(hs1, src, dst, z128)
    hs2 = _mm_narrow(p1, hs1, b1.reshape(hid // 128, 1, 128), dinv, W2, n)
    p2 = prop_narrow(hs2, src, dst, z32)
    out = _final(p2, hs2, b2.reshape(2, 1, 32), dinv, n)
    return out
